# baseline (speedup 1.0000x reference)
import functools

import jax
import jax.numpy as jnp
from jax import lax
from jax.experimental import pallas as pl
from jax.experimental.pallas import tpu as pltpu
from jax.experimental.pallas import tpu_sc as plsc

M = 8192
K = 8192
D = 32
TOK_TILE = 256

_NUM_CORES = 2
_NUM_SUBCORES = 16
_NW = _NUM_CORES * _NUM_SUBCORES
_ROWS_PER_TEC = M // _NW
_IDX_CHUNK = 128


def _dist_body(hi_ref, cb_ref, cn_ref, idx_ref):
    i = pl.program_id(0)
    dims = (((1,), (0,)), ((), ()))
    dot = lax.dot_general(hi_ref[...], cb_ref[...], dims,
                          preferred_element_type=jnp.float32)
    scores = dot + cn_ref[...]
    idx_ref[0, 0, :] = jnp.argmin(scores, axis=1).astype(jnp.int32)


def _tc_dist_argmin(hi, cb, cn):
    grid = (M // TOK_TILE,)
    return pl.pallas_call(
        _dist_body,
        grid=grid,
        in_specs=[
            pl.BlockSpec((TOK_TILE, D), lambda i: (i, 0)),
            pl.BlockSpec((D, K), lambda i: (0, 0)),
            pl.BlockSpec((1, K), lambda i: (0, 0)),
        ],
        out_specs=[
            pl.BlockSpec((1, 1, TOK_TILE), lambda i: (i, 0, 0)),
        ],
        out_shape=[
            jax.ShapeDtypeStruct((M // TOK_TILE, 1, TOK_TILE), jnp.int32),
        ],
    )(hi, cb, cn)


def _sc_gather_body(table_hbm, idx_hbm, flat_hbm, out_hbm, psum_hbm,
                    idx_v, rows_v, flat_v, acc_v, sem, fsem):
    wid = lax.axis_index("s") * _NUM_CORES + lax.axis_index("c")
    base = wid * _ROWS_PER_TEC
    n_chunks = _ROWS_PER_TEC // _IDX_CHUNK
    fcopy = pltpu.async_copy(
        flat_hbm.at[pl.ds(base, _ROWS_PER_TEC)], flat_v, fsem)
    for j in range(n_chunks):
        pltpu.sync_copy(
            idx_hbm.at[pl.ds(base + j * _IDX_CHUNK, _IDX_CHUNK)], idx_v.at[j]
        )
    copies = [
        pltpu.async_copy(
            table_hbm.at[idx_v.at[j]],
            rows_v.at[pl.ds(j * _IDX_CHUNK, _IDX_CHUNK)],
            sem,
        )
        for j in range(n_chunks)
    ]
    for c in copies:
        c.wait()
    fcopy.wait()
    out_copy = pltpu.async_copy(
        rows_v, out_hbm.at[pl.ds(base, _ROWS_PER_TEC)], sem)
    acc = jnp.zeros((16,), jnp.float32)
    for r in range(_ROWS_PER_TEC):
        for h in range(0, D, 16):
            dlt = flat_v[r, pl.ds(h, 16)] - rows_v[r, pl.ds(h, 16)]
            acc = acc + dlt * dlt
    acc_v[...] = acc
    out_copy.wait()
    pltpu.sync_copy(acc_v, psum_hbm.at[wid])


@functools.cache
def _sc_gather():
    return pl.kernel(
        _sc_gather_body,
        out_type=(jax.ShapeDtypeStruct((M, D), jnp.float32),
                  jax.ShapeDtypeStruct((_NW, 16), jnp.float32)),
        mesh=plsc.VectorSubcoreMesh(core_axis_name="c", subcore_axis_name="s"),
        scratch_types=[
            pltpu.VMEM((_ROWS_PER_TEC // _IDX_CHUNK, _IDX_CHUNK), jnp.int32),
            pltpu.VMEM((_ROWS_PER_TEC, D), jnp.float32),
            pltpu.VMEM((_ROWS_PER_TEC, D), jnp.float32),
            pltpu.VMEM((16,), jnp.float32),
            pltpu.SemaphoreType.DMA,
            pltpu.SemaphoreType.DMA,
        ],
        compiler_params=pltpu.CompilerParams(use_tc_tiling_on_sc=False),
    )


def kernel(z, codebook):
    B, N, d = z.shape
    flat = z.reshape(-1, d)
    hi = flat.astype(jnp.bfloat16)
    cbt = (-2.0 * codebook.T).astype(jnp.bfloat16)
    cn = jnp.sum(codebook * codebook, axis=1)[None, :]
    (idx3,) = _tc_dist_argmin(hi, cbt, cn)
    idx = idx3.reshape(M)
    quant, psum = _sc_gather()(codebook, idx, flat)
    commit_loss = (0.25 / (M * d)) * jnp.sum(psum)
    return (quant.reshape(B, N, d), idx.reshape(B, N), commit_loss)


# speedup vs baseline: 1.0215x; 1.0215x over previous
import functools

import jax
import jax.numpy as jnp
from jax import lax
from jax.experimental import pallas as pl
from jax.experimental.pallas import tpu as pltpu
from jax.experimental.pallas import tpu_sc as plsc

M = 8192
K = 8192
D = 32
TOK_TILE = 1024

_NUM_CORES = 2
_NUM_SUBCORES = 16
_NW = _NUM_CORES * _NUM_SUBCORES
_ROWS_PER_TEC = M // _NW
_IDX_CHUNK = 128


def _dist_body(hi_ref, cb_ref, cn_ref, idx_ref):
    i = pl.program_id(0)
    dims = (((1,), (0,)), ((), ()))
    dot = lax.dot_general(hi_ref[...], cb_ref[...], dims,
                          preferred_element_type=jnp.float32)
    scores = dot + cn_ref[...]
    idx_ref[0, 0, :] = jnp.argmin(scores, axis=1).astype(jnp.int32)


def _tc_dist_argmin(hi, cb, cn):
    grid = (M // TOK_TILE,)
    return pl.pallas_call(
        _dist_body,
        grid=grid,
        in_specs=[
            pl.BlockSpec((TOK_TILE, D), lambda i: (i, 0)),
            pl.BlockSpec((D, K), lambda i: (0, 0)),
            pl.BlockSpec((1, K), lambda i: (0, 0)),
        ],
        out_specs=[
            pl.BlockSpec((1, 1, TOK_TILE), lambda i: (i, 0, 0)),
        ],
        out_shape=[
            jax.ShapeDtypeStruct((M // TOK_TILE, 1, TOK_TILE), jnp.int32),
        ],
    )(hi, cb, cn)


def _sc_gather_body(table_hbm, idx_hbm, flat_hbm, out_hbm, psum_hbm,
                    idx_v, rows_v, flat_v, acc_v, sem, fsem):
    wid = lax.axis_index("s") * _NUM_CORES + lax.axis_index("c")
    base = wid * _ROWS_PER_TEC
    n_chunks = _ROWS_PER_TEC // _IDX_CHUNK
    fcopy = pltpu.async_copy(
        flat_hbm.at[pl.ds(base, _ROWS_PER_TEC)], flat_v, fsem)
    for j in range(n_chunks):
        pltpu.sync_copy(
            idx_hbm.at[pl.ds(base + j * _IDX_CHUNK, _IDX_CHUNK)], idx_v.at[j]
        )
    copies = [
        pltpu.async_copy(
            table_hbm.at[idx_v.at[j]],
            rows_v.at[pl.ds(j * _IDX_CHUNK, _IDX_CHUNK)],
            sem,
        )
        for j in range(n_chunks)
    ]
    for c in copies:
        c.wait()
    fcopy.wait()
    out_copy = pltpu.async_copy(
        rows_v, out_hbm.at[pl.ds(base, _ROWS_PER_TEC)], sem)
    acc = jnp.zeros((16,), jnp.float32)
    for r in range(_ROWS_PER_TEC):
        for h in range(0, D, 16):
            dlt = flat_v[r, pl.ds(h, 16)] - rows_v[r, pl.ds(h, 16)]
            acc = acc + dlt * dlt
    acc_v[...] = acc
    out_copy.wait()
    pltpu.sync_copy(acc_v, psum_hbm.at[wid])


@functools.cache
def _sc_gather():
    return pl.kernel(
        _sc_gather_body,
        out_type=(jax.ShapeDtypeStruct((M, D), jnp.float32),
                  jax.ShapeDtypeStruct((_NW, 16), jnp.float32)),
        mesh=plsc.VectorSubcoreMesh(core_axis_name="c", subcore_axis_name="s"),
        scratch_types=[
            pltpu.VMEM((_ROWS_PER_TEC // _IDX_CHUNK, _IDX_CHUNK), jnp.int32),
            pltpu.VMEM((_ROWS_PER_TEC, D), jnp.float32),
            pltpu.VMEM((_ROWS_PER_TEC, D), jnp.float32),
            pltpu.VMEM((16,), jnp.float32),
            pltpu.SemaphoreType.DMA,
            pltpu.SemaphoreType.DMA,
        ],
        compiler_params=pltpu.CompilerParams(use_tc_tiling_on_sc=False),
    )


def kernel(z, codebook):
    B, N, d = z.shape
    flat = z.reshape(-1, d)
    hi = flat.astype(jnp.bfloat16)
    cbt = (-2.0 * codebook.T).astype(jnp.bfloat16)
    cn = jnp.sum(codebook * codebook, axis=1)[None, :]
    (idx3,) = _tc_dist_argmin(hi, cbt, cn)
    idx = idx3.reshape(M)
    quant, psum = _sc_gather()(codebook, idx, flat)
    commit_loss = (0.25 / (M * d)) * jnp.sum(psum)
    return (quant.reshape(B, N, d), idx.reshape(B, N), commit_loss)


# speedup vs baseline: 1.0352x; 1.0134x over previous
import functools

import jax
import jax.numpy as jnp
from jax import lax
from jax.experimental import pallas as pl
from jax.experimental.pallas import tpu as pltpu
from jax.experimental.pallas import tpu_sc as plsc

M = 8192
K = 8192
D = 32
TOK_TILE = 512

_NUM_CORES = 2
_NUM_SUBCORES = 16
_NW = _NUM_CORES * _NUM_SUBCORES
_ROWS_PER_TEC = M // _NW
_IDX_CHUNK = 128


def _dist_body(hi_ref, cb_ref, cn_ref, idx_ref):
    i = pl.program_id(0)
    dims = (((1,), (0,)), ((), ()))
    dot = lax.dot_general(hi_ref[...], cb_ref[...], dims,
                          preferred_element_type=jnp.float32)
    scores = dot + cn_ref[...]
    idx_ref[0, 0, :] = jnp.argmin(scores, axis=1).astype(jnp.int32)


def _tc_dist_argmin(hi, cb, cn):
    grid = (M // TOK_TILE,)
    return pl.pallas_call(
        _dist_body,
        grid=grid,
        in_specs=[
            pl.BlockSpec((TOK_TILE, D), lambda i: (i, 0)),
            pl.BlockSpec((D, K), lambda i: (0, 0)),
            pl.BlockSpec((1, K), lambda i: (0, 0)),
        ],
        out_specs=[
            pl.BlockSpec((1, 1, TOK_TILE), lambda i: (i, 0, 0)),
        ],
        out_shape=[
            jax.ShapeDtypeStruct((M // TOK_TILE, 1, TOK_TILE), jnp.int32),
        ],
    )(hi, cb, cn)


def _sc_gather_body(table_hbm, idx_hbm, flat_hbm, out_hbm, psum_hbm,
                    idx_v, rows_v, flat_v, acc_v, sem, fsem):
    wid = lax.axis_index("s") * _NUM_CORES + lax.axis_index("c")
    base = wid * _ROWS_PER_TEC
    n_chunks = _ROWS_PER_TEC // _IDX_CHUNK
    fcopy = pltpu.async_copy(
        flat_hbm.at[pl.ds(base, _ROWS_PER_TEC)], flat_v, fsem)
    for j in range(n_chunks):
        pltpu.sync_copy(
            idx_hbm.at[pl.ds(base + j * _IDX_CHUNK, _IDX_CHUNK)], idx_v.at[j]
        )
    copies = [
        pltpu.async_copy(
            table_hbm.at[idx_v.at[j]],
            rows_v.at[pl.ds(j * _IDX_CHUNK, _IDX_CHUNK)],
            sem,
        )
        for j in range(n_chunks)
    ]
    for c in copies:
        c.wait()
    fcopy.wait()
    out_copy = pltpu.async_copy(
        rows_v, out_hbm.at[pl.ds(base, _ROWS_PER_TEC)], sem)
    acc = jnp.zeros((16,), jnp.float32)
    for r in range(_ROWS_PER_TEC):
        for h in range(0, D, 16):
            dlt = flat_v[r, pl.ds(h, 16)] - rows_v[r, pl.ds(h, 16)]
            acc = acc + dlt * dlt
    acc_v[...] = acc
    out_copy.wait()
    pltpu.sync_copy(acc_v, psum_hbm.at[wid])


@functools.cache
def _sc_gather():
    return pl.kernel(
        _sc_gather_body,
        out_type=(jax.ShapeDtypeStruct((M, D), jnp.float32),
                  jax.ShapeDtypeStruct((_NW, 16), jnp.float32)),
        mesh=plsc.VectorSubcoreMesh(core_axis_name="c", subcore_axis_name="s"),
        scratch_types=[
            pltpu.VMEM((_ROWS_PER_TEC // _IDX_CHUNK, _IDX_CHUNK), jnp.int32),
            pltpu.VMEM((_ROWS_PER_TEC, D), jnp.float32),
            pltpu.VMEM((_ROWS_PER_TEC, D), jnp.float32),
            pltpu.VMEM((16,), jnp.float32),
            pltpu.SemaphoreType.DMA,
            pltpu.SemaphoreType.DMA,
        ],
        compiler_params=pltpu.CompilerParams(use_tc_tiling_on_sc=False),
    )


def kernel(z, codebook):
    B, N, d = z.shape
    flat = z.reshape(-1, d)
    hi = flat.astype(jnp.bfloat16)
    cbt = (-2.0 * codebook.T).astype(jnp.bfloat16)
    cn = jnp.sum(codebook * codebook, axis=1)[None, :]
    (idx3,) = _tc_dist_argmin(hi, cbt, cn)
    idx = idx3.reshape(M)
    quant, psum = _sc_gather()(codebook, idx, flat)
    commit_loss = (0.25 / (M * d)) * jnp.sum(psum)
    return (quant.reshape(B, N, d), idx.reshape(B, N), commit_loss)
